# baseline (speedup 1.0000x reference)
import jax
import jax.numpy as jnp
from jax import lax
from jax.experimental import pallas as pl
from jax.experimental.pallas import tpu as pltpu
from jax.experimental.pallas import tpu_sc as plsc

_B = 4
_L = 8192
_D = 1024

_info = plsc.get_sparse_core_info()
_NC = _info.num_cores
_NS = _info.num_subcores
_NW = _NC * _NS
_ROWS_PER_W = _L // _NW
_CHUNK = 64
_NCHUNK = _ROWS_PER_W // _CHUNK


def _copy_body(table_hbm, out_hbm, sem):
    wid = lax.axis_index("s") * _NC + lax.axis_index("c")
    base = wid * _ROWS_PER_W
    copies = [
        pltpu.async_copy(
            table_hbm.at[pl.ds(base, _ROWS_PER_W)],
            out_hbm.at[b, pl.ds(base, _ROWS_PER_W)],
            sem,
        )
        for b in range(_B)
    ]
    for c in copies:
        c.wait()


def kernel(x, table):
    del x
    mesh = plsc.VectorSubcoreMesh(core_axis_name="c", subcore_axis_name="s")
    run = pl.kernel(
        _copy_body,
        mesh=mesh,
        out_type=jax.ShapeDtypeStruct((_B, _L, _D), jnp.float32),
        scratch_types=[
            pltpu.SemaphoreType.DMA,
        ],
    )
    return run(table)


# speedup vs baseline: 54.2500x; 54.2500x over previous
import jax
import jax.numpy as jnp
from jax import lax
from jax.experimental import pallas as pl
from jax.experimental.pallas import tpu as pltpu
from jax.experimental.pallas import tpu_sc as plsc

_B = 4
_L = 8192
_D = 1024

_info = plsc.get_sparse_core_info()
_NC = _info.num_cores
_NS = _info.num_subcores
_NW = _NC * _NS
_ROWS_PER_W = _L // _NW
_CHUNK = 32
_NCHUNK = _ROWS_PER_W // _CHUNK


def _copy_body(table_hbm, out_hbm, buf0, buf1, ld0, ld1, st0, st1):
    wid = lax.axis_index("s") * _NC + lax.axis_index("c")
    base = wid * _ROWS_PER_W
    bufs = (buf0, buf1)
    ld_sems = (ld0, ld1)
    st_sems = (st0, st1)
    pending_stores = [[], []]
    loads = [None] * _NCHUNK
    loads[0] = pltpu.async_copy(table_hbm.at[pl.ds(base, _CHUNK)], bufs[0], ld_sems[0])
    for i in range(_NCHUNK):
        if i + 1 < _NCHUNK:
            nb = (i + 1) % 2
            for c in pending_stores[nb]:
                c.wait()
            pending_stores[nb] = []
            loads[i + 1] = pltpu.async_copy(
                table_hbm.at[pl.ds(base + (i + 1) * _CHUNK, _CHUNK)],
                bufs[nb],
                ld_sems[nb],
            )
        loads[i].wait()
        cb = i % 2
        row0 = base + i * _CHUNK
        for b in range(_B):
            pending_stores[cb].append(
                pltpu.async_copy(
                    bufs[cb], out_hbm.at[b, pl.ds(row0, _CHUNK)], st_sems[cb]
                )
            )
    for lst in pending_stores:
        for c in lst:
            c.wait()


def kernel(x, table):
    del x
    mesh = plsc.VectorSubcoreMesh(core_axis_name="c", subcore_axis_name="s")
    run = pl.kernel(
        _copy_body,
        mesh=mesh,
        out_type=jax.ShapeDtypeStruct((_B, _L, _D), jnp.float32),
        scratch_types=[
            pltpu.VMEM((_CHUNK, _D), jnp.float32),
            pltpu.VMEM((_CHUNK, _D), jnp.float32),
            pltpu.SemaphoreType.DMA,
            pltpu.SemaphoreType.DMA,
            pltpu.SemaphoreType.DMA,
            pltpu.SemaphoreType.DMA,
        ],
    )
    return run(table)


# speedup vs baseline: 72.5190x; 1.3368x over previous
import jax
import jax.numpy as jnp
from jax.experimental import pallas as pl
from jax.experimental.pallas import tpu as pltpu

_B = 4
_L = 8192
_D = 1024
_BLK = 256


def _tc_body(table_ref, out_ref):
    out_ref[...] = jnp.broadcast_to(table_ref[...][None], (_B, _BLK, _D))


def kernel(x, table):
    del x
    return pl.pallas_call(
        _tc_body,
        grid=(_L // _BLK,),
        in_specs=[pl.BlockSpec((_BLK, _D), lambda j: (j, 0))],
        out_specs=pl.BlockSpec((_B, _BLK, _D), lambda j: (0, j, 0)),
        out_shape=jax.ShapeDtypeStruct((_B, _L, _D), jnp.float32),
    )(table)
